# Initial kernel scaffold; baseline (speedup 1.0000x reference)
import jax
import jax.numpy as jnp
from jax.experimental import pallas as pl


def kernel(x, router_w, router_b, Wg, Wu, Wd, Wg_s, Wu_s, Wd_s):
    raise NotImplementedError("write your pallas kernel here")


# speedup vs baseline: 2.3862x; 2.3862x over previous
import functools

import jax
import jax.numpy as jnp
from jax import lax
from jax.experimental import pallas as pl
from jax.experimental.pallas import tpu as pltpu

E = 64
K = 8
D = 768
F = 256
FS = 1536
T = 256

CHUNK_M = 64
NCHUNK = T // CHUNK_M
FS_CHUNK = 384


def _router_kernel(x_ref, rw_ref, rb_ref, maskT_ref, rankT_ref, nums_ref):
    x = x_ref[...]
    logits = lax.dot_general(x, rw_ref[...], (((1,), (1,)), ((), ())),
                             preferred_element_type=jnp.float32)
    logits = logits + rb_ref[...]
    m = jnp.max(logits, axis=1, keepdims=True)
    ex = jnp.exp(logits - m)
    p = ex / jnp.sum(ex, axis=1, keepdims=True)

    li = lax.broadcasted_iota(jnp.int32, (T, E), 1)
    mask = jnp.zeros((T, E), dtype=jnp.float32)
    for _ in range(K):
        cur = jnp.max(p, axis=1, keepdims=True)
        cand = jnp.where(p == cur, li, E)
        first = jnp.min(cand, axis=1, keepdims=True)
        sel = li == first
        mask = mask + jnp.where(sel, p, 0.0)
        p = jnp.where(sel, -1.0, p)
    mask = mask / jnp.sum(mask, axis=1, keepdims=True)

    ind = (mask > 0).astype(jnp.float32)
    ti = lax.broadcasted_iota(jnp.int32, (T, T), 0)
    tj = lax.broadcasted_iota(jnp.int32, (T, T), 1)
    lower = (tj < ti).astype(jnp.float32)
    rank = lax.dot_general(lower, ind, (((1,), (0,)), ((), ())),
                           preferred_element_type=jnp.float32)

    maskT_ref[...] = mask.T
    rankT_ref[...] = rank.T.astype(jnp.int32)
    nums_ref[...] = jnp.sum(ind, axis=0, keepdims=True).astype(jnp.int32)


def _moe_kernel(nums_ref, maskT_ref, rankT_ref, x_ref, wg_ref, wu_ref, wd_ref,
                out_ref):
    e = pl.program_id(0)
    c = pl.program_id(1)

    @pl.when(jnp.logical_and(e == 0, c == 0))
    def _():
        out_ref[...] = jnp.zeros_like(out_ref)

    @pl.when(c * CHUNK_M < nums_ref[e])
    def _():
        mrow = maskT_ref[...].reshape(1, T)
        rrow = rankT_ref[...].reshape(1, T)
        rid = lax.broadcasted_iota(jnp.int32, (CHUNK_M, T), 0) + c * CHUNK_M
        sel = jnp.logical_and(rrow == rid, mrow > 0)
        selw = sel.astype(jnp.float32) * mrow
        xi = lax.dot_general(selw, x_ref[...], (((1,), (0,)), ((), ())),
                             preferred_element_type=jnp.float32)
        wg = wg_ref[...].reshape(F, D)
        wu = wu_ref[...].reshape(F, D)
        wd = wd_ref[...].reshape(D, F)
        g = lax.dot_general(xi, wg, (((1,), (1,)), ((), ())),
                            preferred_element_type=jnp.float32)
        u = lax.dot_general(xi, wu, (((1,), (1,)), ((), ())),
                            preferred_element_type=jnp.float32)
        h = g * jax.nn.sigmoid(g) * u
        y = lax.dot_general(h, wd, (((1,), (1,)), ((), ())),
                            preferred_element_type=jnp.float32)
        out_ref[...] += lax.dot_general(selw, y, (((0,), (0,)), ((), ())),
                                        preferred_element_type=jnp.float32)


def _shared_kernel(x_ref, wgs_ref, wus_ref, wds_ref, out_ref):
    i = pl.program_id(0)
    x = x_ref[...]
    g = lax.dot_general(x, wgs_ref[...], (((1,), (1,)), ((), ())),
                        preferred_element_type=jnp.float32)
    u = lax.dot_general(x, wus_ref[...], (((1,), (1,)), ((), ())),
                        preferred_element_type=jnp.float32)
    h = g * jax.nn.sigmoid(g) * u
    y = lax.dot_general(h, wds_ref[...], (((1,), (1,)), ((), ())),
                        preferred_element_type=jnp.float32)

    @pl.when(i == 0)
    def _():
        out_ref[...] = jnp.zeros_like(out_ref)

    out_ref[...] += y


def kernel(x, router_w, router_b, Wg, Wu, Wd, Wg_s, Wu_s, Wd_s):
    maskT, rankT, nums = pl.pallas_call(
        _router_kernel,
        out_shape=(
            jax.ShapeDtypeStruct((E, T), jnp.float32),
            jax.ShapeDtypeStruct((E, T), jnp.int32),
            jax.ShapeDtypeStruct((1, E), jnp.int32),
        ),
    )(x, router_w, router_b.reshape(1, E))

    maskT3 = maskT.reshape(E, 1, T)
    rankT3 = rankT.reshape(E, 1, T)
    nums1 = nums.reshape(E)

    grid_spec = pltpu.PrefetchScalarGridSpec(
        num_scalar_prefetch=1,
        grid=(E, NCHUNK),
        in_specs=[
            pl.BlockSpec((1, 1, T), lambda e, c, nums: (e, 0, 0)),
            pl.BlockSpec((1, 1, T), lambda e, c, nums: (e, 0, 0)),
            pl.BlockSpec((T, D), lambda e, c, nums: (0, 0)),
            pl.BlockSpec((1, F, D), lambda e, c, nums: (e, 0, 0)),
            pl.BlockSpec((1, F, D), lambda e, c, nums: (e, 0, 0)),
            pl.BlockSpec((1, D, F), lambda e, c, nums: (e, 0, 0)),
        ],
        out_specs=pl.BlockSpec((T, D), lambda e, c, nums: (0, 0)),
    )
    moe_out = pl.pallas_call(
        _moe_kernel,
        grid_spec=grid_spec,
        out_shape=jax.ShapeDtypeStruct((T, D), jnp.float32),
    )(nums1, maskT3, rankT3, x, Wg, Wu, Wd)

    shared_out = pl.pallas_call(
        _shared_kernel,
        grid=(FS // FS_CHUNK,),
        in_specs=[
            pl.BlockSpec((T, D), lambda i: (0, 0)),
            pl.BlockSpec((FS_CHUNK, D), lambda i: (i, 0)),
            pl.BlockSpec((FS_CHUNK, D), lambda i: (i, 0)),
            pl.BlockSpec((D, FS_CHUNK), lambda i: (0, i)),
        ],
        out_specs=pl.BlockSpec((T, D), lambda i: (0, 0)),
        out_shape=jax.ShapeDtypeStruct((T, D), jnp.float32),
    )(x, Wg_s, Wu_s, Wd_s)

    return moe_out + shared_out
